# baseline (speedup 1.0000x reference)
import jax
import jax.numpy as jnp
from jax.experimental import pallas as pl

EMBED = 768
EXPERTS = 8
TOK_BLK = 2048


def _router_body(x_ref, w_ref, o_ref):
    o_ref[...] = jnp.dot(x_ref[...], w_ref[...],
                         preferred_element_type=jnp.float32)


def kernel(x, W):
    B, S, D = x.shape
    N = B * S
    xf = x.reshape(N, D)
    out = pl.pallas_call(
        _router_body,
        grid=(N // TOK_BLK,),
        in_specs=[
            pl.BlockSpec((TOK_BLK, D), lambda i: (i, 0)),
            pl.BlockSpec((D, EXPERTS), lambda i: (0, 0)),
        ],
        out_specs=pl.BlockSpec((TOK_BLK, EXPERTS), lambda i: (i, 0)),
        out_shape=jax.ShapeDtypeStruct((N, EXPERTS), jnp.float32),
    )(xf, W)
    return out.reshape(B, S, EXPERTS)
